# Initial kernel scaffold; baseline (speedup 1.0000x reference)
import jax
import jax.numpy as jnp
from jax.experimental import pallas as pl


def kernel(x, ln_g, ln_b, gW1, gb1, gW2, gb2, eW1, eb1, eW2, eb2):
    raise NotImplementedError("write your pallas kernel here")


# speedup vs baseline: 1.0166x; 1.0166x over previous
import functools

import jax
import jax.numpy as jnp
from jax.experimental import pallas as pl
from jax.experimental.pallas import tpu as pltpu

NUM_EXPERTS = 64
NUM_GROUPS = 8
EPG = NUM_EXPERTS // NUM_GROUPS
TOP_K = 2


def _body(x_ref, lng_ref, lnb_ref, gW1_ref, gb1_ref, gW2_ref, gb2_ref,
          eW1_ref, eb1_ref, eW2_ref, eb2_ref,
          idx_out_ref, p_out_ref, aux_out_ref,
          xn_ref, gp_ref, probs_ref, acc_ref,
          *, n_tok_blocks, tb, inv_n):
    t = pl.program_id(0)
    g = pl.program_id(1)
    f32 = jnp.float32
    bf16 = jnp.bfloat16

    @pl.when(g == 0)
    def _group_pass():
        xb = x_ref[:, :]
        mu = jnp.mean(xb, axis=1, keepdims=True)
        var = jnp.mean((xb - mu) ** 2, axis=1, keepdims=True)
        xn = (xb - mu) / jnp.sqrt(var + 1e-5) * lng_ref[:, :] + lnb_ref[:, :]
        xnb = xn.astype(bf16)
        xn_ref[:, :] = xnb
        h = jax.nn.relu(
            jnp.dot(xnb, gW1_ref[:, :], preferred_element_type=f32)
            + gb1_ref[:, :])
        gl = (jnp.dot(h.astype(bf16), gW2_ref[:, :], preferred_element_type=f32)
              + gb2_ref[:, :])
        m = jnp.max(gl, axis=1, keepdims=True)
        e = jnp.exp(gl - m)
        gp_ref[:, :] = e / jnp.sum(e, axis=1, keepdims=True)

    xnb = xn_ref[:, :]
    eh = jax.nn.relu(
        jnp.dot(xnb, eW1_ref[0], preferred_element_type=f32) + eb1_ref[0])
    el = (jnp.dot(eh.astype(bf16), eW2_ref[0], preferred_element_type=f32)
          + eb2_ref[0])
    m = jnp.max(el, axis=1, keepdims=True)
    e = jnp.exp(el - m)
    ep = e / jnp.sum(e, axis=1, keepdims=True)
    probs_ref[g] = ep

    @pl.when(g == NUM_GROUPS - 1)
    def _finalize_block():
        gp = gp_ref[:, :]
        cols = [probs_ref[gg] * gp[:, gg:gg + 1] for gg in range(NUM_GROUPS)]
        p = jnp.concatenate(cols, axis=1)
        iota = jax.lax.broadcasted_iota(jnp.int32, (tb, NUM_EXPERTS), 1)
        m1 = jnp.max(p, axis=1, keepdims=True)
        i1 = jnp.min(jnp.where(p == m1, iota, NUM_EXPERTS), axis=1,
                     keepdims=True)
        pm = jnp.where(iota == i1, -1.0, p)
        m2 = jnp.max(pm, axis=1, keepdims=True)
        i2 = jnp.min(jnp.where(pm == m2, iota, NUM_EXPERTS), axis=1,
                     keepdims=True)
        s = m1 + m2
        idx_out_ref[:, :] = jnp.concatenate([i1, i2], axis=1)
        p_out_ref[:, :] = jnp.concatenate([m1 / s, m2 / s], axis=1)

        colsum = jnp.sum(p, axis=0, keepdims=True)

        @pl.when(t == 0)
        def _():
            acc_ref[:, :] = colsum

        @pl.when(t != 0)
        def _():
            acc_ref[:, :] = acc_ref[:, :] + colsum

        @pl.when(t == n_tok_blocks - 1)
        def _aux():
            pbar = acc_ref[:, :] * inv_n
            aux_out_ref[:, :] = jnp.sum(
                pbar * jnp.log(pbar * NUM_EXPERTS + 1e-9),
                axis=1, keepdims=True)


def kernel(x, ln_g, ln_b, gW1, gb1, gW2, gb2, eW1, eb1, eW2, eb2):
    B, S, D = x.shape
    G = NUM_GROUPS
    H2 = eW1.shape[2]
    N = B * S
    TB = 512 if N % 512 == 0 else N
    n_tok_blocks = N // TB

    bf16 = jnp.bfloat16
    x2 = x.reshape(N, D)
    gW1b = gW1.astype(bf16)
    gW2b = gW2.astype(bf16)
    eW1b = eW1.astype(bf16)
    eW2b = eW2.astype(bf16)

    grid = (n_tok_blocks, G)
    body = functools.partial(_body, n_tok_blocks=n_tok_blocks, tb=TB,
                             inv_n=1.0 / N)
    out = pl.pallas_call(
        body,
        grid=grid,
        in_specs=[
            pl.BlockSpec((TB, D), lambda t, g: (t, 0)),
            pl.BlockSpec((1, D), lambda t, g: (0, 0)),
            pl.BlockSpec((1, D), lambda t, g: (0, 0)),
            pl.BlockSpec((D, D), lambda t, g: (0, 0)),
            pl.BlockSpec((1, D), lambda t, g: (0, 0)),
            pl.BlockSpec((D, G), lambda t, g: (0, 0)),
            pl.BlockSpec((1, G), lambda t, g: (0, 0)),
            pl.BlockSpec((1, D, H2), lambda t, g: (g, 0, 0)),
            pl.BlockSpec((1, 1, H2), lambda t, g: (g, 0, 0)),
            pl.BlockSpec((1, H2, EPG), lambda t, g: (g, 0, 0)),
            pl.BlockSpec((1, 1, EPG), lambda t, g: (g, 0, 0)),
        ],
        out_specs=[
            pl.BlockSpec((TB, TOP_K), lambda t, g: (t, 0)),
            pl.BlockSpec((TB, TOP_K), lambda t, g: (t, 0)),
            pl.BlockSpec((1, 1), lambda t, g: (0, 0)),
        ],
        out_shape=[
            jax.ShapeDtypeStruct((N, TOP_K), jnp.int32),
            jax.ShapeDtypeStruct((N, TOP_K), jnp.float32),
            jax.ShapeDtypeStruct((1, 1), jnp.float32),
        ],
        scratch_shapes=[
            pltpu.VMEM((TB, D), bf16),
            pltpu.VMEM((TB, G), jnp.float32),
            pltpu.VMEM((G, TB, EPG), jnp.float32),
            pltpu.VMEM((1, NUM_EXPERTS), jnp.float32),
        ],
        compiler_params=pltpu.CompilerParams(
            dimension_semantics=("arbitrary", "arbitrary")),
    )(x2, ln_g.reshape(1, D), ln_b.reshape(1, D), gW1b,
      gb1.reshape(1, D), gW2b, gb2.reshape(1, G),
      eW1b, eb1.reshape(G, 1, H2), eW2b, eb2.reshape(G, 1, EPG))

    top_k_indices = out[0].reshape(B, S, TOP_K)
    top_k_probs = out[1].reshape(B, S, TOP_K)
    aux_loss = out[2].reshape(())
    return (top_k_indices, top_k_probs, aux_loss)


# speedup vs baseline: 1.1008x; 1.0828x over previous
import functools

import jax
import jax.numpy as jnp
from jax.experimental import pallas as pl
from jax.experimental.pallas import tpu as pltpu

NUM_EXPERTS = 64
NUM_GROUPS = 8
EPG = NUM_EXPERTS // NUM_GROUPS
TOP_K = 2


def _body(x_ref, lng_ref, lnb_ref, gW1_ref, gb1_ref, gW2_ref, gb2_ref,
          eW1_ref, eb1_ref, eW2_ref, eb2_ref,
          idx_out_ref, p_out_ref, aux_out_ref,
          xn_ref, gp_ref, probs_ref, acc_ref,
          *, n_tok_blocks, tb, inv_n):
    t = pl.program_id(0)
    g = pl.program_id(1)
    f32 = jnp.float32
    bf16 = jnp.bfloat16

    @pl.when(g == 0)
    def _group_pass():
        xb = x_ref[:, :]
        mu = jnp.mean(xb, axis=1, keepdims=True)
        var = jnp.mean((xb - mu) ** 2, axis=1, keepdims=True)
        xn = (xb - mu) / jnp.sqrt(var + 1e-5) * lng_ref[:, :] + lnb_ref[:, :]
        xnb = xn.astype(bf16)
        xn_ref[:, :] = xnb
        h = jax.nn.relu(
            jnp.dot(xnb, gW1_ref[:, :], preferred_element_type=f32)
            + gb1_ref[:, :])
        gl = (jnp.dot(h.astype(bf16), gW2_ref[:, :], preferred_element_type=f32)
              + gb2_ref[:, :])
        m = jnp.max(gl, axis=1, keepdims=True)
        e = jnp.exp(gl - m)
        gp_ref[:, :] = e / jnp.sum(e, axis=1, keepdims=True)

    xnb = xn_ref[:, :]
    eh = jax.nn.relu(
        jnp.dot(xnb, eW1_ref[0], preferred_element_type=f32) + eb1_ref[0])
    el = (jnp.dot(eh.astype(bf16), eW2_ref[0], preferred_element_type=f32)
          + eb2_ref[0])
    m = jnp.max(el, axis=1, keepdims=True)
    e = jnp.exp(el - m)
    ep = e / jnp.sum(e, axis=1, keepdims=True)
    probs_ref[g] = ep

    @pl.when(g == NUM_GROUPS - 1)
    def _finalize_block():
        gp = gp_ref[:, :]
        cols = [probs_ref[gg] * gp[:, gg:gg + 1] for gg in range(NUM_GROUPS)]
        p = jnp.concatenate(cols, axis=1)
        iota = jax.lax.broadcasted_iota(jnp.int32, (tb, NUM_EXPERTS), 1)
        m1 = jnp.max(p, axis=1, keepdims=True)
        i1 = jnp.min(jnp.where(p == m1, iota, NUM_EXPERTS), axis=1,
                     keepdims=True)
        pm = jnp.where(iota == i1, -1.0, p)
        m2 = jnp.max(pm, axis=1, keepdims=True)
        i2 = jnp.min(jnp.where(pm == m2, iota, NUM_EXPERTS), axis=1,
                     keepdims=True)
        s = m1 + m2
        idx_out_ref[:, :] = jnp.concatenate([i1, i2], axis=1)
        p_out_ref[:, :] = jnp.concatenate([m1 / s, m2 / s], axis=1)

        colsum = jnp.sum(p, axis=0, keepdims=True)

        @pl.when(t == 0)
        def _():
            acc_ref[:, :] = colsum

        @pl.when(t != 0)
        def _():
            acc_ref[:, :] = acc_ref[:, :] + colsum

        @pl.when(t == n_tok_blocks - 1)
        def _aux():
            pbar = acc_ref[:, :] * inv_n
            aux_out_ref[:, :] = jnp.sum(
                pbar * jnp.log(pbar * NUM_EXPERTS + 1e-9),
                axis=1, keepdims=True)


def kernel(x, ln_g, ln_b, gW1, gb1, gW2, gb2, eW1, eb1, eW2, eb2):
    B, S, D = x.shape
    G = NUM_GROUPS
    H2 = eW1.shape[2]
    N = B * S
    TB = 1024 if N % 1024 == 0 else N
    n_tok_blocks = N // TB

    bf16 = jnp.bfloat16
    x2 = x.reshape(N, D)
    gW1b = gW1.astype(bf16)
    gW2b = gW2.astype(bf16)
    eW1b = eW1.astype(bf16)
    eW2b = eW2.astype(bf16)

    grid = (n_tok_blocks, G)
    body = functools.partial(_body, n_tok_blocks=n_tok_blocks, tb=TB,
                             inv_n=1.0 / N)
    out = pl.pallas_call(
        body,
        grid=grid,
        in_specs=[
            pl.BlockSpec((TB, D), lambda t, g: (t, 0)),
            pl.BlockSpec((1, D), lambda t, g: (0, 0)),
            pl.BlockSpec((1, D), lambda t, g: (0, 0)),
            pl.BlockSpec((D, D), lambda t, g: (0, 0)),
            pl.BlockSpec((1, D), lambda t, g: (0, 0)),
            pl.BlockSpec((D, G), lambda t, g: (0, 0)),
            pl.BlockSpec((1, G), lambda t, g: (0, 0)),
            pl.BlockSpec((1, D, H2), lambda t, g: (g, 0, 0)),
            pl.BlockSpec((1, 1, H2), lambda t, g: (g, 0, 0)),
            pl.BlockSpec((1, H2, EPG), lambda t, g: (g, 0, 0)),
            pl.BlockSpec((1, 1, EPG), lambda t, g: (g, 0, 0)),
        ],
        out_specs=[
            pl.BlockSpec((TB, TOP_K), lambda t, g: (t, 0)),
            pl.BlockSpec((TB, TOP_K), lambda t, g: (t, 0)),
            pl.BlockSpec((1, 1), lambda t, g: (0, 0)),
        ],
        out_shape=[
            jax.ShapeDtypeStruct((N, TOP_K), jnp.int32),
            jax.ShapeDtypeStruct((N, TOP_K), jnp.float32),
            jax.ShapeDtypeStruct((1, 1), jnp.float32),
        ],
        scratch_shapes=[
            pltpu.VMEM((TB, D), bf16),
            pltpu.VMEM((TB, G), jnp.float32),
            pltpu.VMEM((G, TB, EPG), jnp.float32),
            pltpu.VMEM((1, NUM_EXPERTS), jnp.float32),
        ],
        compiler_params=pltpu.CompilerParams(
            dimension_semantics=("arbitrary", "arbitrary")),
    )(x2, ln_g.reshape(1, D), ln_b.reshape(1, D), gW1b,
      gb1.reshape(1, D), gW2b, gb2.reshape(1, G),
      eW1b, eb1.reshape(G, 1, H2), eW2b, eb2.reshape(G, 1, EPG))

    top_k_indices = out[0].reshape(B, S, TOP_K)
    top_k_probs = out[1].reshape(B, S, TOP_K)
    aux_loss = out[2].reshape(())
    return (top_k_indices, top_k_probs, aux_loss)


# speedup vs baseline: 1.1329x; 1.0292x over previous
import functools

import jax
import jax.numpy as jnp
from jax.experimental import pallas as pl
from jax.experimental.pallas import tpu as pltpu

NUM_EXPERTS = 64
NUM_GROUPS = 8
EPG = NUM_EXPERTS // NUM_GROUPS
TOP_K = 2
GPS = 1


def _ln_group_body(x_ref, lng_ref, lnb_ref, gW1_ref, gb1_ref, gW2_ref,
                   gb2_ref, xn_out_ref, gp_out_ref, *, tb, chunks):
    f32 = jnp.float32
    bf16 = jnp.bfloat16
    c = tb // chunks
    gls = []
    for i in range(chunks):
        sl = pl.ds(i * c, c)
        xb = x_ref[sl, :]
        mu = jnp.mean(xb, axis=1, keepdims=True)
        var = jnp.mean((xb - mu) ** 2, axis=1, keepdims=True)
        xn = (xb - mu) / jnp.sqrt(var + 1e-5) * lng_ref[:, :] + lnb_ref[:, :]
        xnb = xn.astype(bf16)
        xn_out_ref[sl, :] = xnb
        h = jax.nn.relu(
            jnp.dot(xnb, gW1_ref[:, :], preferred_element_type=f32)
            + gb1_ref[:, :])
        gls.append(
            jnp.dot(h.astype(bf16), gW2_ref[:, :], preferred_element_type=f32)
            + gb2_ref[:, :])
    for i in range(chunks):
        sl = pl.ds(i * c, c)
        gl = gls[i]
        m = jnp.max(gl, axis=1, keepdims=True)
        e = jnp.exp(gl - m)
        gp_out_ref[sl, :] = e / jnp.sum(e, axis=1, keepdims=True)


def _experts_body(xn_ref, gpprev_ref, eW1_ref, eb1_ref, eW2_ref, eb2_ref,
                  idx_out_ref, p_out_ref, aux_out_ref,
                  probs_ref, acc_ref, *, n_ts, tsb, inv_n):
    ts = pl.program_id(0)
    gs = pl.program_id(1)
    f32 = jnp.float32
    bf16 = jnp.bfloat16
    slot = jax.lax.rem(ts, 2)
    pslot = jax.lax.rem(ts + 1, 2)

    half = tsb // 2

    @pl.when(ts < n_ts)
    def _mm():
        for j in range(GPS):
            for r in range(2):
                sl = pl.ds(r * half, half)
                eh = jax.nn.relu(
                    jnp.dot(xn_ref[sl, :], eW1_ref[j],
                            preferred_element_type=f32)
                    + eb1_ref[j])
                el = (jnp.dot(eh.astype(bf16), eW2_ref[j],
                              preferred_element_type=f32) + eb2_ref[j])
                m = jnp.max(el, axis=1, keepdims=True)
                e = jnp.exp(el - m)
                probs_ref[slot * NUM_GROUPS + gs * GPS + j, sl, :] = (
                    e / jnp.sum(e, axis=1, keepdims=True))

    @pl.when(jnp.logical_and(ts > 0, gs == 0))
    def _epilogue():
        gp = gpprev_ref[:, :]
        cols = [probs_ref[pslot * NUM_GROUPS + gg] * gp[:, gg:gg + 1]
                for gg in range(NUM_GROUPS)]
        p = jnp.concatenate(cols, axis=1)
        iota = jax.lax.broadcasted_iota(jnp.int32, (tsb, NUM_EXPERTS), 1)
        m1 = jnp.max(p, axis=1, keepdims=True)
        i1 = jnp.min(jnp.where(p == m1, iota, NUM_EXPERTS), axis=1,
                     keepdims=True)
        pm = jnp.where(iota == i1, -1.0, p)
        m2 = jnp.max(pm, axis=1, keepdims=True)
        i2 = jnp.min(jnp.where(pm == m2, iota, NUM_EXPERTS), axis=1,
                     keepdims=True)
        s = m1 + m2
        idx_out_ref[:, :] = jnp.concatenate([i1, i2], axis=1)
        p_out_ref[:, :] = jnp.concatenate([m1 / s, m2 / s], axis=1)

        colsum = jnp.sum(p, axis=0, keepdims=True)

        @pl.when(ts == 1)
        def _():
            acc_ref[:, :] = colsum

        @pl.when(ts != 1)
        def _():
            acc_ref[:, :] = acc_ref[:, :] + colsum

        @pl.when(ts == n_ts)
        def _aux():
            pbar = acc_ref[:, :] * inv_n
            aux_out_ref[:, :] = jnp.sum(
                pbar * jnp.log(pbar * NUM_EXPERTS + 1e-9),
                axis=1, keepdims=True)


def kernel(x, ln_g, ln_b, gW1, gb1, gW2, gb2, eW1, eb1, eW2, eb2):
    B, S, D = x.shape
    G = NUM_GROUPS
    H2 = eW1.shape[2]
    N = B * S
    TB = 512 if N % 512 == 0 else N
    n_tb = N // TB
    TSB = 2048 if N % 2048 == 0 else N
    n_ts = N // TSB

    bf16 = jnp.bfloat16
    x2 = x.reshape(N, D)
    gW1b = gW1.astype(bf16)
    gW2b = gW2.astype(bf16)
    eW1b = eW1.astype(bf16)
    eW2b = eW2.astype(bf16)

    bodyA = functools.partial(_ln_group_body, tb=TB, chunks=2)
    xnb, gp = pl.pallas_call(
        bodyA,
        grid=(n_tb,),
        in_specs=[
            pl.BlockSpec((TB, D), lambda t: (t, 0)),
            pl.BlockSpec((1, D), lambda t: (0, 0)),
            pl.BlockSpec((1, D), lambda t: (0, 0)),
            pl.BlockSpec((D, D), lambda t: (0, 0)),
            pl.BlockSpec((1, D), lambda t: (0, 0)),
            pl.BlockSpec((D, G), lambda t: (0, 0)),
            pl.BlockSpec((1, G), lambda t: (0, 0)),
        ],
        out_specs=[
            pl.BlockSpec((TB, D), lambda t: (t, 0)),
            pl.BlockSpec((TB, G), lambda t: (t, 0)),
        ],
        out_shape=[
            jax.ShapeDtypeStruct((N, D), bf16),
            jax.ShapeDtypeStruct((N, G), jnp.float32),
        ],
        compiler_params=pltpu.CompilerParams(
            dimension_semantics=("arbitrary",)),
    )(x2, ln_g.reshape(1, D), ln_b.reshape(1, D), gW1b,
      gb1.reshape(1, D), gW2b, gb2.reshape(1, G))

    n_gs = G // GPS
    bodyB = functools.partial(_experts_body, n_ts=n_ts, tsb=TSB, inv_n=1.0 / N)
    out = pl.pallas_call(
        bodyB,
        grid=(n_ts + 1, n_gs),
        in_specs=[
            pl.BlockSpec((TSB, D),
                         lambda ts, gs: (jnp.minimum(ts, n_ts - 1), 0)),
            pl.BlockSpec((TSB, G),
                         lambda ts, gs: (jnp.maximum(ts - 1, 0), 0)),
            pl.BlockSpec((GPS, D, H2), lambda ts, gs: (gs, 0, 0)),
            pl.BlockSpec((GPS, 1, H2), lambda ts, gs: (gs, 0, 0)),
            pl.BlockSpec((GPS, H2, EPG), lambda ts, gs: (gs, 0, 0)),
            pl.BlockSpec((GPS, 1, EPG), lambda ts, gs: (gs, 0, 0)),
        ],
        out_specs=[
            pl.BlockSpec((TSB, TOP_K),
                         lambda ts, gs: (jnp.maximum(ts - 1, 0), 0)),
            pl.BlockSpec((TSB, TOP_K),
                         lambda ts, gs: (jnp.maximum(ts - 1, 0), 0)),
            pl.BlockSpec((1, 1), lambda ts, gs: (0, 0)),
        ],
        out_shape=[
            jax.ShapeDtypeStruct((N, TOP_K), jnp.int32),
            jax.ShapeDtypeStruct((N, TOP_K), jnp.float32),
            jax.ShapeDtypeStruct((1, 1), jnp.float32),
        ],
        scratch_shapes=[
            pltpu.VMEM((2 * G, TSB, EPG), jnp.float32),
            pltpu.VMEM((1, NUM_EXPERTS), jnp.float32),
        ],
        compiler_params=pltpu.CompilerParams(
            dimension_semantics=("arbitrary", "arbitrary")),
    )(xnb, gp, eW1b, eb1.reshape(G, 1, H2), eW2b, eb2.reshape(G, 1, EPG))

    top_k_indices = out[0].reshape(B, S, TOP_K)
    top_k_probs = out[1].reshape(B, S, TOP_K)
    aux_loss = out[2].reshape(())
    return (top_k_indices, top_k_probs, aux_loss)
